# baseline (speedup 1.0000x reference)
import functools

import jax
import jax.numpy as jnp
from jax.experimental import pallas as pl

B, D, C = 16384, 1024, 51


def _head_kernel(x_ref, wt_ref, b_ref, red_ref, probs_ref, val_ref):
    logits = jnp.dot(x_ref[...], wt_ref[...], preferred_element_type=jnp.float32)
    logits = logits + b_ref[...]
    m = jnp.max(logits, axis=1, keepdims=True)
    e = jnp.exp(logits - m)
    r = jnp.dot(e, red_ref[...], preferred_element_type=jnp.float32)
    rinv = 1.0 / r[:, 0:1]
    probs_ref[...] = e * rinv
    val_ref[0, 0, :] = (r[:, 1] * rinv[:, 0])


@jax.jit
def kernel(x, W, b, bins):
    blk = 512
    nb = B // blk
    wt = W.T
    b2 = b.reshape(1, C)
    red = jnp.stack([jnp.ones((C,), jnp.float32), bins], axis=1)
    probs, val = pl.pallas_call(
        _head_kernel,
        grid=(nb,),
        in_specs=[
            pl.BlockSpec((blk, D), lambda i: (i, 0)),
            pl.BlockSpec((D, C), lambda i: (0, 0)),
            pl.BlockSpec((1, C), lambda i: (0, 0)),
            pl.BlockSpec((C, 2), lambda i: (0, 0)),
        ],
        out_specs=[
            pl.BlockSpec((blk, C), lambda i: (i, 0)),
            pl.BlockSpec((1, 1, blk), lambda i: (i, 0, 0)),
        ],
        out_shape=[
            jax.ShapeDtypeStruct((B, C), jnp.float32),
            jax.ShapeDtypeStruct((nb, 1, blk), jnp.float32),
        ],
    )(x, wt, b2, red)
    return probs, val.reshape(B)


# speedup vs baseline: 1.1649x; 1.1649x over previous
import functools

import jax
import jax.numpy as jnp
from jax.experimental import pallas as pl

B, D, C = 16384, 1024, 51


def _head_kernel(x_ref, wt_ref, b_ref, red_ref, probs_ref, val_ref):
    logits = jnp.dot(x_ref[...], wt_ref[...], preferred_element_type=jnp.float32)
    logits = logits + b_ref[...]
    m = jnp.max(logits, axis=1, keepdims=True)
    e = jnp.exp(logits - m)
    r = jnp.dot(e, red_ref[...], preferred_element_type=jnp.float32)
    rinv = 1.0 / r[:, 0:1]
    probs_ref[...] = e * rinv
    val_ref[0, 0, :] = (r[:, 1] * rinv[:, 0])


@jax.jit
def kernel(x, W, b, bins):
    blk = 2048
    nb = B // blk
    wt = W.T
    b2 = b.reshape(1, C)
    red = jnp.stack([jnp.ones((C,), jnp.float32), bins], axis=1)
    probs, val = pl.pallas_call(
        _head_kernel,
        grid=(nb,),
        in_specs=[
            pl.BlockSpec((blk, D), lambda i: (i, 0)),
            pl.BlockSpec((D, C), lambda i: (0, 0)),
            pl.BlockSpec((1, C), lambda i: (0, 0)),
            pl.BlockSpec((C, 2), lambda i: (0, 0)),
        ],
        out_specs=[
            pl.BlockSpec((blk, C), lambda i: (i, 0)),
            pl.BlockSpec((1, 1, blk), lambda i: (i, 0, 0)),
        ],
        out_shape=[
            jax.ShapeDtypeStruct((B, C), jnp.float32),
            jax.ShapeDtypeStruct((nb, 1, blk), jnp.float32),
        ],
    )(x, wt, b2, red)
    return probs, val.reshape(B)


# speedup vs baseline: 1.4324x; 1.2296x over previous
import jax
import jax.numpy as jnp
from jax import lax
from jax.experimental import pallas as pl

B, D, C = 16384, 1024, 51


def _head_kernel(x_ref, w_ref, b_ref, bins_ref, probs_ref, val_ref):
    lt = lax.dot_general(
        w_ref[...], x_ref[...],
        (((1,), (1,)), ((), ())),
        preferred_element_type=jnp.float32,
    )
    lt = lt + b_ref[...]
    m = jnp.max(lt, axis=0, keepdims=True)
    e = jnp.exp(lt - m)
    s = jnp.sum(e, axis=0, keepdims=True)
    rinv = 1.0 / s
    num = jnp.sum(e * bins_ref[...], axis=0, keepdims=True)
    pt = e * rinv
    probs_ref[...] = pt.T
    val_ref[0, 0, :] = (num * rinv)[0, :]


@jax.jit
def kernel(x, W, b, bins):
    blk = 1024
    nb = B // blk
    b2 = b.reshape(C, 1)
    bins2 = bins.reshape(C, 1)
    probs, val = pl.pallas_call(
        _head_kernel,
        grid=(nb,),
        in_specs=[
            pl.BlockSpec((blk, D), lambda i: (i, 0)),
            pl.BlockSpec((C, D), lambda i: (0, 0)),
            pl.BlockSpec((C, 1), lambda i: (0, 0)),
            pl.BlockSpec((C, 1), lambda i: (0, 0)),
        ],
        out_specs=[
            pl.BlockSpec((blk, C), lambda i: (i, 0)),
            pl.BlockSpec((1, 1, blk), lambda i: (i, 0, 0)),
        ],
        out_shape=[
            jax.ShapeDtypeStruct((B, C), jnp.float32),
            jax.ShapeDtypeStruct((nb, 1, blk), jnp.float32),
        ],
    )(x, W, b2, bins2)
    return probs, val.reshape(B)


# speedup vs baseline: 1.5676x; 1.0944x over previous
import jax
import jax.numpy as jnp
from jax import lax
from jax.experimental import pallas as pl

B, D, C = 16384, 1024, 51


def _head(x_ref, w_ref, b_ref, bins_ref, probs_ref, val_ref, row0):
    lt = lax.dot_general(
        w_ref[...], x_ref[...],
        (((1,), (1,)), ((), ())),
        preferred_element_type=jnp.float32,
    )
    lt = lt + b_ref[...]
    m = jnp.max(lt, axis=0, keepdims=True)
    e = jnp.exp(lt - m)
    s = jnp.sum(e, axis=0, keepdims=True)
    rinv = 1.0 / s
    num = jnp.sum(e * bins_ref[...], axis=0, keepdims=True)
    pt = e * rinv
    blk = x_ref.shape[0]
    probs_ref[row0:row0 + blk, :] = pt.T
    val_ref[0, 0, row0:row0 + blk] = (num * rinv)[0, :]


def _head_kernel(x0_ref, x1_ref, w_ref, b_ref, bins_ref, probs_ref, val_ref):
    blk = x0_ref.shape[0]
    _head(x0_ref, w_ref, b_ref, bins_ref, probs_ref, val_ref, 0)
    _head(x1_ref, w_ref, b_ref, bins_ref, probs_ref, val_ref, blk)


@jax.jit
def kernel(x, W, b, bins):
    blk = 1024
    nb = B // blk
    b2 = b.reshape(C, 1)
    bins2 = bins.reshape(C, 1)
    probs, val = pl.pallas_call(
        _head_kernel,
        grid=(nb // 2,),
        in_specs=[
            pl.BlockSpec((blk, D), lambda i: (2 * i, 0)),
            pl.BlockSpec((blk, D), lambda i: (2 * i + 1, 0)),
            pl.BlockSpec((C, D), lambda i: (0, 0)),
            pl.BlockSpec((C, 1), lambda i: (0, 0)),
            pl.BlockSpec((C, 1), lambda i: (0, 0)),
        ],
        out_specs=[
            pl.BlockSpec((2 * blk, C), lambda i: (i, 0)),
            pl.BlockSpec((1, 1, 2 * blk), lambda i: (i, 0, 0)),
        ],
        out_shape=[
            jax.ShapeDtypeStruct((B, C), jnp.float32),
            jax.ShapeDtypeStruct((nb // 2, 1, 2 * blk), jnp.float32),
        ],
    )(x, x, W, b2, bins2)
    return probs, val.reshape(B)
